# baseline (speedup 1.0000x reference)
import functools

import jax
import jax.numpy as jnp
from jax import lax
from jax.experimental import pallas as pl
from jax.experimental.pallas import tpu as pltpu
from jax.experimental.pallas import tpu_sc as plsc

_START = 128
_F = 256
_HF = _F - _START

_NC = 2
_NS = 16
_NW = _NC * _NS
_L = 16

_FB = 256


def _rank_body(out_ref):
    pid = pl.program_id(0)
    row = lax.broadcasted_iota(jnp.uint32, (_FB, _HF), 0)
    col = lax.broadcasted_iota(jnp.uint32, (_FB, _HF), 1)
    p = (pid.astype(jnp.uint32) * _FB + row) * _HF + col
    k0 = jnp.uint32(0)
    k1 = jnp.uint32(1234)
    ks2 = jnp.uint32(0 ^ 1234 ^ 0x1BD11BDA)
    x0 = jnp.zeros_like(p) + k0
    x1 = p + k1
    rot = ((13, 15, 26, 6), (17, 29, 16, 24))
    ks = (k0, k1, ks2)
    for i in range(5):
        for d in rot[i % 2]:
            x0 = x0 + x1
            x1 = ((x1 << jnp.uint32(d)) | (x1 >> jnp.uint32(32 - d))) ^ x0
        x0 = x0 + ks[(i + 1) % 3]
        x1 = x1 + ks[(i + 2) % 3] + jnp.uint32(i + 1)
    bits = x0 ^ x1
    key = (((bits >> jnp.uint32(9)) << jnp.uint32(7)) | col).astype(jnp.int32)
    acc = jnp.zeros((_FB, _HF), jnp.int32)
    for k in range(_HF):
        kk = key[:, k:k + 1]
        acc = acc + (kk < key).astype(jnp.int32)
    out_ref[...] = acc


def _tc_ranks(n_frames):
    return pl.pallas_call(
        _rank_body,
        grid=(n_frames // _FB,),
        out_specs=pl.BlockSpec((_FB, _HF), lambda i: (i, 0)),
        out_shape=jax.ShapeDtypeStruct((n_frames, _HF), jnp.int32),
    )()


def _sc_permute(x_flat, rank_flat, n_frames):
    frames_pw = n_frames // _NW
    ch = 128
    n_chunks = frames_pw // ch
    mesh = plsc.VectorSubcoreMesh(core_axis_name="c", subcore_axis_name="s")

    @functools.partial(
        pl.kernel,
        out_type=jax.ShapeDtypeStruct((n_frames * _F,), jnp.float32),
        mesh=mesh,
        compiler_params=pltpu.CompilerParams(needs_layout_passes=False),
        scratch_types=[
            pltpu.VMEM((ch * _F,), jnp.float32),
            pltpu.VMEM((ch * _HF,), jnp.int32),
        ],
    )
    def k(x_hbm, rank_hbm, out_hbm, xv, idxv):
        wid = lax.axis_index("s") * _NC + lax.axis_index("c")
        for c in range(n_chunks):
            frame0 = (wid * n_chunks + c) * ch
            pltpu.sync_copy(x_hbm.at[pl.ds(frame0 * _F, ch * _F)], xv)
            pltpu.sync_copy(rank_hbm.at[pl.ds(frame0 * _HF, ch * _HF)], idxv)

            def body(f, carry):
                hb = f * _F + _START
                src = []
                dst = []
                for s in range(_HF // _L):
                    src.append(xv[pl.ds(hb + s * _L, _L)])
                    dst.append(idxv[pl.ds(f * _HF + s * _L, _L)] + hb)
                for s in range(_HF // _L):
                    plsc.store_scatter(xv, [dst[s]], src[s])
                return carry

            lax.fori_loop(0, ch, body, 0)
            pltpu.sync_copy(xv, out_hbm.at[pl.ds(frame0 * _F, ch * _F)])

    return k(x_flat, rank_flat)


def kernel(x):
    B, T, F = x.shape
    ranks = _tc_ranks(B * T)
    out = _sc_permute(x.reshape(-1), ranks.reshape(-1), B * T)
    return out.reshape(B, T, F)


# speedup vs baseline: 1.6644x; 1.6644x over previous
import functools

import jax
import jax.numpy as jnp
from jax import lax
from jax.experimental import pallas as pl
from jax.experimental.pallas import tpu as pltpu
from jax.experimental.pallas import tpu_sc as plsc

_START = 128
_F = 256
_HF = _F - _START

_NC = 2
_NS = 16
_NW = _NC * _NS
_L = 16

_FB = 128


def _rank_body(out_ref, key_ref):
    pid = pl.program_id(0)
    binv = lax.broadcasted_iota(jnp.uint32, (_HF, _FB), 0)
    frame = lax.broadcasted_iota(jnp.uint32, (_HF, _FB), 1)
    p = (pid.astype(jnp.uint32) * _FB + frame) * _HF + binv
    k0 = jnp.uint32(0)
    k1 = jnp.uint32(1234)
    ks2 = jnp.uint32(0 ^ 1234 ^ 0x1BD11BDA)
    x0 = jnp.zeros_like(p) + k0
    x1 = p + k1
    rot = ((13, 15, 26, 6), (17, 29, 16, 24))
    ks = (k0, k1, ks2)
    for i in range(5):
        for d in rot[i % 2]:
            x0 = x0 + x1
            x1 = ((x1 << jnp.uint32(d)) | (x1 >> jnp.uint32(32 - d))) ^ x0
        x0 = x0 + ks[(i + 1) % 3]
        x1 = x1 + ks[(i + 2) % 3] + jnp.uint32(i + 1)
    bits = x0 ^ x1
    key = (((bits >> jnp.uint32(9)) << jnp.uint32(7)) | binv).astype(jnp.int32)
    key_ref[...] = key
    acc = jnp.zeros((_HF, _FB), jnp.int32)
    for k in range(_HF):
        kk = key_ref[k:k + 1, :]
        acc = acc + (kk < key).astype(jnp.int32)
    out_ref[0] = acc


def _tc_ranks(n_frames):
    nb = n_frames // _FB
    return pl.pallas_call(
        _rank_body,
        grid=(nb,),
        out_specs=pl.BlockSpec((1, _HF, _FB), lambda i: (i, 0, 0)),
        out_shape=jax.ShapeDtypeStruct((nb, _HF, _FB), jnp.int32),
        scratch_shapes=[pltpu.VMEM((_HF, _FB), jnp.int32)],
    )()


def _sc_permute(x_flat, rank_flat, n_frames):
    frames_pw = n_frames // _NW
    ch = _FB
    n_chunks = frames_pw // ch
    mesh = plsc.VectorSubcoreMesh(core_axis_name="c", subcore_axis_name="s")

    @functools.partial(
        pl.kernel,
        out_type=jax.ShapeDtypeStruct((n_frames * _F,), jnp.float32),
        mesh=mesh,
        compiler_params=pltpu.CompilerParams(needs_layout_passes=False),
        scratch_types=[
            pltpu.VMEM((ch * _F,), jnp.float32),
            pltpu.VMEM((_HF * ch,), jnp.int32),
        ],
    )
    def k(x_hbm, rank_hbm, out_hbm, xv, idxv):
        wid = lax.axis_index("s") * _NC + lax.axis_index("c")
        colbase = [
            (jnp.arange(s * _L, (s + 1) * _L, dtype=jnp.int32)) * ch
            for s in range(_HF // _L)
        ]
        for c in range(n_chunks):
            blk = wid * n_chunks + c
            frame0 = blk * ch
            pltpu.sync_copy(x_hbm.at[pl.ds(frame0 * _F, ch * _F)], xv)
            pltpu.sync_copy(rank_hbm.at[pl.ds(blk * _HF * ch, _HF * ch)], idxv)

            def body(f, carry):
                hb = f * _F + _START
                src = []
                dst = []
                for s in range(_HF // _L):
                    src.append(xv[pl.ds(hb + s * _L, _L)])
                    rank = plsc.load_gather(idxv, [colbase[s] + f])
                    dst.append(rank + hb)
                for s in range(_HF // _L):
                    plsc.store_scatter(xv, [dst[s]], src[s])
                return carry

            lax.fori_loop(0, ch, body, 0)
            pltpu.sync_copy(xv, out_hbm.at[pl.ds(frame0 * _F, ch * _F)])

    return k(x_flat, rank_flat)


def kernel(x):
    B, T, F = x.shape
    ranks = _tc_ranks(B * T)
    out = _sc_permute(x.reshape(-1), ranks.reshape(-1), B * T)
    return out.reshape(B, T, F)


# speedup vs baseline: 1.9212x; 1.1543x over previous
import functools

import jax
import jax.numpy as jnp
from jax import lax
from jax.experimental import pallas as pl
from jax.experimental.pallas import tpu as pltpu
from jax.experimental.pallas import tpu_sc as plsc

_START = 128
_F = 256
_HF = _F - _START

_NC = 2
_NS = 16
_NW = _NC * _NS
_L = 16

_FB = 128


def _rank_body(out_ref, key_ref):
    pid = pl.program_id(0)
    binv = lax.broadcasted_iota(jnp.uint32, (_HF, _FB), 0)
    frame = lax.broadcasted_iota(jnp.uint32, (_HF, _FB), 1)
    p = (pid.astype(jnp.uint32) * _FB + frame) * _HF + binv
    k0 = jnp.uint32(0)
    k1 = jnp.uint32(1234)
    ks2 = jnp.uint32(0 ^ 1234 ^ 0x1BD11BDA)
    x0 = jnp.zeros_like(p) + k0
    x1 = p + k1
    rot = ((13, 15, 26, 6), (17, 29, 16, 24))
    ks = (k0, k1, ks2)
    for i in range(5):
        for d in rot[i % 2]:
            x0 = x0 + x1
            x1 = ((x1 << jnp.uint32(d)) | (x1 >> jnp.uint32(32 - d))) ^ x0
        x0 = x0 + ks[(i + 1) % 3]
        x1 = x1 + ks[(i + 2) % 3] + jnp.uint32(i + 1)
    bits = x0 ^ x1
    key = (((bits >> jnp.uint32(9)) << jnp.uint32(7)) | binv).astype(jnp.int32)
    key_ref[...] = key
    acc = jnp.zeros((_HF, _FB), jnp.int32)
    for k in range(_HF):
        kk = key_ref[k:k + 1, :]
        acc = acc + (kk < key).astype(jnp.int32)
    out_ref[...] = acc.T


def _tc_ranks(n_frames):
    return pl.pallas_call(
        _rank_body,
        grid=(n_frames // _FB,),
        out_specs=pl.BlockSpec((_FB, _HF), lambda i: (i, 0)),
        out_shape=jax.ShapeDtypeStruct((n_frames, _HF), jnp.int32),
        scratch_shapes=[pltpu.VMEM((_HF, _FB), jnp.int32)],
    )()


def _sc_permute(x_flat, rank_flat, n_frames):
    frames_pw = n_frames // _NW
    ch = _FB
    n_chunks = frames_pw // ch
    mesh = plsc.VectorSubcoreMesh(core_axis_name="c", subcore_axis_name="s")

    @functools.partial(
        pl.kernel,
        out_type=jax.ShapeDtypeStruct((n_frames * _F,), jnp.float32),
        mesh=mesh,
        compiler_params=pltpu.CompilerParams(needs_layout_passes=False),
        scratch_types=[
            pltpu.VMEM((ch * _F,), jnp.float32),
            pltpu.VMEM((_HF * ch,), jnp.int32),
        ],
    )
    def k(x_hbm, rank_hbm, out_hbm, xv, idxv):
        wid = lax.axis_index("s") * _NC + lax.axis_index("c")
        for c in range(n_chunks):
            frame0 = (wid * n_chunks + c) * ch
            pltpu.sync_copy(x_hbm.at[pl.ds(frame0 * _F, ch * _F)], xv)
            pltpu.sync_copy(rank_hbm.at[pl.ds(frame0 * _HF, ch * _HF)], idxv)

            def body(f, carry):
                hb = f * _F + _START
                src = []
                dst = []
                for s in range(_HF // _L):
                    src.append(xv[pl.ds(hb + s * _L, _L)])
                    dst.append(idxv[pl.ds(f * _HF + s * _L, _L)] + hb)
                for s in range(_HF // _L):
                    plsc.store_scatter(xv, [dst[s]], src[s])
                return carry

            lax.fori_loop(0, ch, body, 0)
            pltpu.sync_copy(xv, out_hbm.at[pl.ds(frame0 * _F, ch * _F)])

    return k(x_flat, rank_flat)


def kernel(x):
    B, T, F = x.shape
    ranks = _tc_ranks(B * T)
    out = _sc_permute(x.reshape(-1), ranks.reshape(-1), B * T)
    return out.reshape(B, T, F)


# speedup vs baseline: 1.9371x; 1.0083x over previous
import functools

import jax
import jax.numpy as jnp
from jax import lax
from jax.experimental import pallas as pl
from jax.experimental.pallas import tpu as pltpu
from jax.experimental.pallas import tpu_sc as plsc

_START = 128
_F = 256
_HF = _F - _START

_NC = 2
_NS = 16
_NW = _NC * _NS
_L = 16

_FB = 128
_SLABS = 4


def _rank_body(f00, out_ref, key_ref):
    pid = pl.program_id(0)
    binv = lax.broadcasted_iota(jnp.uint32, (_HF, _FB), 0)
    frame = lax.broadcasted_iota(jnp.uint32, (_HF, _FB), 1)
    p = (jnp.uint32(f00) + pid.astype(jnp.uint32) * _FB + frame) * _HF + binv
    k0 = jnp.uint32(0)
    k1 = jnp.uint32(1234)
    ks2 = jnp.uint32(0 ^ 1234 ^ 0x1BD11BDA)
    x0 = jnp.zeros_like(p) + k0
    x1 = p + k1
    rot = ((13, 15, 26, 6), (17, 29, 16, 24))
    ks = (k0, k1, ks2)
    for i in range(5):
        for d in rot[i % 2]:
            x0 = x0 + x1
            x1 = ((x1 << jnp.uint32(d)) | (x1 >> jnp.uint32(32 - d))) ^ x0
        x0 = x0 + ks[(i + 1) % 3]
        x1 = x1 + ks[(i + 2) % 3] + jnp.uint32(i + 1)
    bits = x0 ^ x1
    key = (((bits >> jnp.uint32(9)) << jnp.uint32(7)) | binv).astype(jnp.int32)
    key_ref[...] = key
    acc = jnp.zeros((_HF, _FB), jnp.int32)
    for k in range(_HF):
        kk = key_ref[k:k + 1, :]
        acc = acc + (kk < key).astype(jnp.int32)
    out_ref[...] = acc.T


def _tc_ranks(n_frames, f00):
    return pl.pallas_call(
        functools.partial(_rank_body, f00),
        grid=(n_frames // _FB,),
        out_specs=pl.BlockSpec((_FB, _HF), lambda i: (i, 0)),
        out_shape=jax.ShapeDtypeStruct((n_frames, _HF), jnp.int32),
        scratch_shapes=[pltpu.VMEM((_HF, _FB), jnp.int32)],
        name=f"ranks_{f00}",
    )()


def _sc_permute(x_flat, rank_flat, n_frames, tag):
    frames_pw = n_frames // _NW
    ch = _FB
    n_chunks = frames_pw // ch
    mesh = plsc.VectorSubcoreMesh(core_axis_name="c", subcore_axis_name="s")

    @functools.partial(
        pl.kernel,
        out_type=jax.ShapeDtypeStruct((n_frames * _F,), jnp.float32),
        mesh=mesh,
        compiler_params=pltpu.CompilerParams(needs_layout_passes=False),
        scratch_types=[
            pltpu.VMEM((ch * _F,), jnp.float32),
            pltpu.VMEM((ch * _HF,), jnp.int32),
        ],
        name=f"scperm_{tag}",
    )
    def k(x_hbm, rank_hbm, out_hbm, xv, idxv):
        wid = lax.axis_index("s") * _NC + lax.axis_index("c")
        for c in range(n_chunks):
            frame0 = (wid * n_chunks + c) * ch
            pltpu.sync_copy(x_hbm.at[pl.ds(frame0 * _F, ch * _F)], xv)
            pltpu.sync_copy(rank_hbm.at[pl.ds(frame0 * _HF, ch * _HF)], idxv)

            def body(f, carry):
                hb = f * _F + _START
                src = []
                dst = []
                for s in range(_HF // _L):
                    src.append(xv[pl.ds(hb + s * _L, _L)])
                    dst.append(idxv[pl.ds(f * _HF + s * _L, _L)] + hb)
                for s in range(_HF // _L):
                    plsc.store_scatter(xv, [dst[s]], src[s])
                return carry

            lax.fori_loop(0, ch, body, 0)
            pltpu.sync_copy(xv, out_hbm.at[pl.ds(frame0 * _F, ch * _F)])

    return k(x_flat, rank_flat)


def kernel(x):
    B, T, F = x.shape
    n = B * T
    per = n // _SLABS
    xf = x.reshape(-1)
    outs = []
    for s in range(_SLABS):
        ranks = _tc_ranks(per, s * per)
        outs.append(
            _sc_permute(xf[s * per * _F:(s + 1) * per * _F],
                        ranks.reshape(-1), per, s))
    return jnp.concatenate(outs).reshape(B, T, F)


# speedup vs baseline: 2.7242x; 1.4064x over previous
import functools

import jax
import jax.numpy as jnp
from jax import lax
from jax.experimental import pallas as pl
from jax.experimental.pallas import tpu as pltpu
from jax.experimental.pallas import tpu_sc as plsc

_START = 128
_F = 256
_HF = _F - _START

_NC = 2
_NS = 16
_NW = _NC * _NS
_L = 16

_FB = 256
_SLABS = 4

_IOTA16 = tuple(range(16))


def _keys_body(f00, out_ref):
    pid = pl.program_id(0)
    frame = lax.broadcasted_iota(jnp.uint32, (_FB, _HF), 0)
    col = lax.broadcasted_iota(jnp.uint32, (_FB, _HF), 1)
    p = (jnp.uint32(f00) + pid.astype(jnp.uint32) * _FB + frame) * _HF + col
    k0 = jnp.uint32(0)
    k1 = jnp.uint32(1234)
    ks2 = jnp.uint32(0 ^ 1234 ^ 0x1BD11BDA)
    x0 = jnp.zeros_like(p) + k0
    x1 = p + k1
    rot = ((13, 15, 26, 6), (17, 29, 16, 24))
    ks = (k0, k1, ks2)
    for i in range(5):
        for d in rot[i % 2]:
            x0 = x0 + x1
            x1 = ((x1 << jnp.uint32(d)) | (x1 >> jnp.uint32(32 - d))) ^ x0
        x0 = x0 + ks[(i + 1) % 3]
        x1 = x1 + ks[(i + 2) % 3] + jnp.uint32(i + 1)
    bits = x0 ^ x1
    out_ref[...] = (((bits >> jnp.uint32(9)) << jnp.uint32(7))
                    | col).astype(jnp.int32)


def _tc_keys(n_frames, f00):
    return pl.pallas_call(
        functools.partial(_keys_body, f00),
        grid=(n_frames // _FB,),
        out_specs=pl.BlockSpec((_FB, _HF), lambda i: (i, 0)),
        out_shape=jax.ShapeDtypeStruct((n_frames, _HF), jnp.int32),
        name=f"keys_{f00}",
    )()


def _take(v, idx):
    return jnp.take_along_axis(v, idx, axis=0, mode="promise_in_bounds")


def _clean16(v):
    iota = lax.broadcasted_iota(jnp.int32, (_L,), 0)
    for d in (8, 4, 2, 1):
        p = _take(v, iota ^ d)
        bitd = (iota & d) != 0
        v = jnp.where(bitd, jnp.maximum(v, p), jnp.minimum(v, p))
    return v


def _clean_list(vs):
    m = len(vs)
    d = m // 2
    while d >= 1:
        for base in range(0, m, 2 * d):
            for i in range(base, base + d):
                a, b = vs[i], vs[i + d]
                vs[i], vs[i + d] = jnp.minimum(a, b), jnp.maximum(a, b)
        d //= 2
    return [_clean16(v) for v in vs]


def _merge(a, b):
    brev = [lax.rev(v, (0,)) for v in b[::-1]]
    lo = [jnp.minimum(x, y) for x, y in zip(a, brev)]
    hi = [jnp.maximum(x, y) for x, y in zip(a, brev)]
    return _clean_list(lo) + _clean_list(hi)


def _sc_sortperm(x_flat, key_flat, n_frames, tag):
    frames_pw = n_frames // _NW
    ch = 128
    n_chunks = frames_pw // ch
    mesh = plsc.VectorSubcoreMesh(core_axis_name="c", subcore_axis_name="s")

    @functools.partial(
        pl.kernel,
        out_type=jax.ShapeDtypeStruct((n_frames * _F,), jnp.float32),
        mesh=mesh,
        compiler_params=pltpu.CompilerParams(needs_layout_passes=False),
        scratch_types=[
            pltpu.VMEM((ch * _F,), jnp.float32),
            pltpu.VMEM((ch * _HF,), jnp.int32),
        ],
        name=f"scperm_{tag}",
    )
    def k(x_hbm, key_hbm, out_hbm, xv, kv):
        wid = lax.axis_index("s") * _NC + lax.axis_index("c")
        for c in range(n_chunks):
            frame0 = (wid * n_chunks + c) * ch
            pltpu.sync_copy(x_hbm.at[pl.ds(frame0 * _F, ch * _F)], xv)
            pltpu.sync_copy(key_hbm.at[pl.ds(frame0 * _HF, ch * _HF)], kv)

            def body(f, carry):
                hb = f * _F + _START
                runs = [
                    [lax.sort(kv[pl.ds(f * _HF + s * _L, _L)])]
                    for s in range(_HF // _L)
                ]
                l1 = [_merge(runs[2 * i], runs[2 * i + 1]) for i in range(4)]
                l2 = [_merge(l1[0], l1[1]), _merge(l1[2], l1[3])]
                l3 = _merge(l2[0], l2[1])
                vals = [
                    plsc.load_gather(xv, [(v & 127) + hb]) for v in l3
                ]
                for s in range(_HF // _L):
                    xv[pl.ds(hb + s * _L, _L)] = vals[s]
                return carry

            lax.fori_loop(0, ch, body, 0)
            pltpu.sync_copy(xv, out_hbm.at[pl.ds(frame0 * _F, ch * _F)])

    return k(x_flat, key_flat)


def kernel(x):
    B, T, F = x.shape
    n = B * T
    per = n // _SLABS
    xf = x.reshape(-1)
    outs = []
    for s in range(_SLABS):
        keys = _tc_keys(per, s * per)
        outs.append(
            _sc_sortperm(xf[s * per * _F:(s + 1) * per * _F],
                         keys.reshape(-1), per, s))
    return jnp.concatenate(outs).reshape(B, T, F)
